# baseline (speedup 1.0000x reference)
import functools

import jax
import jax.numpy as jnp
from jax import lax
from jax.experimental import pallas as pl
from jax.experimental.pallas import tpu as pltpu
from jax.experimental.pallas import tpu_sc as plsc

EMBED = 1024
HID = 4096
NE = 8
K = 2
S = 2048
P = S * K
BLOCK = 128
NB = P // BLOCK + NE
NPAD = NB * BLOCK
NEG = -1e30

TB = 256


def _router_body(x_ref, w1_ref, b1_ref, w2_ref, b2_ref,
                 te_ref, tg_ref, cnt_ref):
    i = pl.program_id(0)
    h = jnp.maximum(
        jnp.dot(x_ref[...], w1_ref[...], preferred_element_type=jnp.float32)
        + b1_ref[...], 0.0)
    s = jnp.dot(h, w2_ref[...], preferred_element_type=jnp.float32) + b2_ref[...]
    iota = lax.broadcasted_iota(jnp.int32, (TB, NE), 1)
    m1 = jnp.max(s, axis=1, keepdims=True)
    i1 = jnp.min(jnp.where(s == m1, iota, NE), axis=1, keepdims=True)
    s2 = jnp.where(iota == i1, NEG, s)
    m2 = jnp.max(s2, axis=1, keepdims=True)
    i2 = jnp.min(jnp.where(s2 == m2, iota, NE), axis=1, keepdims=True)
    g1 = 1.0 / (1.0 + jnp.exp(m2 - m1))
    te_ref[...] = jnp.concatenate([i1, i2], axis=1)
    tg_ref[...] = jnp.concatenate([g1, 1.0 - g1], axis=1)
    one = (iota == i1).astype(jnp.int32) + (iota == i2).astype(jnp.int32)
    cadd = jnp.sum(one, axis=0, keepdims=True)

    @pl.when(i == 0)
    def _():
        cnt_ref[...] = cadd

    @pl.when(i > 0)
    def _():
        cnt_ref[...] = cnt_ref[...] + cadd


def _router(x2, Wr1, br1, Wr2, br2):
    return pl.pallas_call(
        _router_body,
        grid=(S // TB,),
        in_specs=[
            pl.BlockSpec((TB, EMBED), lambda i: (i, 0)),
            pl.BlockSpec((EMBED, HID), lambda i: (0, 0)),
            pl.BlockSpec((1, HID), lambda i: (0, 0)),
            pl.BlockSpec((HID, NE), lambda i: (0, 0)),
            pl.BlockSpec((1, NE), lambda i: (0, 0)),
        ],
        out_specs=[
            pl.BlockSpec((TB, K), lambda i: (i, 0)),
            pl.BlockSpec((TB, K), lambda i: (i, 0)),
            pl.BlockSpec((1, NE), lambda i: (0, 0)),
        ],
        out_shape=[
            jax.ShapeDtypeStruct((S, K), jnp.int32),
            jax.ShapeDtypeStruct((S, K), jnp.float32),
            jax.ShapeDtypeStruct((1, NE), jnp.int32),
        ],
    )(x2, Wr1, br1.reshape(1, HID), Wr2, br2.reshape(1, NE))


_MESH = plsc.VectorSubcoreMesh(core_axis_name="c", subcore_axis_name="s")
_SC_PARAMS = pltpu.CompilerParams(needs_layout_passes=False)


@functools.partial(
    pl.kernel,
    out_type=[
        jax.ShapeDtypeStruct((NPAD,), jnp.int32),
        jax.ShapeDtypeStruct((NPAD,), jnp.float32),
        jax.ShapeDtypeStruct((P,), jnp.int32),
    ],
    mesh=_MESH,
    compiler_params=_SC_PARAMS,
    scratch_types=[
        pltpu.VMEM((P,), jnp.int32),
        pltpu.VMEM((P,), jnp.float32),
        pltpu.VMEM((NPAD + 16,), jnp.int32),
        pltpu.VMEM((NPAD + 16,), jnp.float32),
        pltpu.VMEM((P,), jnp.int32),
    ],
)
def _compact(key_hbm, g_hbm, srt_hbm, gsrt_hbm, pos_hbm,
             key_v, g_v, srt_v, gsrt_v, pos_v):
    wid = lax.axis_index("s") * 2 + lax.axis_index("c")

    @pl.when(wid == 0)
    def _():
        pltpu.sync_copy(key_hbm, key_v)
        pltpu.sync_copy(g_hbm, g_v)
        zi = jnp.zeros((16,), jnp.int32)
        zf = jnp.zeros((16,), jnp.float32)

        def zero_body(i, _):
            srt_v[pl.ds(i * 16, 16)] = zi
            gsrt_v[pl.ds(i * 16, 16)] = zf
            return 0

        lax.fori_loop(0, NPAD // 16, zero_body, 0)
        lane = lax.iota(jnp.int32, 16)

        ptr = jnp.int32(0)
        for e in range(NE):
            def chunk(c, ptr, e=e):
                pair = c * 16 + lane
                k = key_v[pl.ds(c * 16, 16)]
                g = g_v[pl.ds(c * 16, 16)]
                m = k == e
                cum = jnp.cumsum(m.astype(jnp.int32))
                slots = jnp.where(m, ptr + cum - 1, NPAD + lane)
                plsc.store_scatter(srt_v, [slots], pair >> 1)
                plsc.store_scatter(gsrt_v, [slots], g)
                old = pos_v[pl.ds(c * 16, 16)]
                pos_v[pl.ds(c * 16, 16)] = jnp.where(m, slots, old)
                return ptr + cum[15]

            ptr = lax.fori_loop(0, P // 16, chunk, ptr)
            ptr = ((ptr + BLOCK - 1) // BLOCK) * BLOCK

        pltpu.sync_copy(srt_v.at[pl.ds(0, NPAD)], srt_hbm)
        pltpu.sync_copy(gsrt_v.at[pl.ds(0, NPAD)], gsrt_hbm)
        pltpu.sync_copy(pos_v, pos_hbm)


def _expert_body(be_ref, nb_ref, x2_ref, srt_ref, w1_ref, b1_ref, w2_ref,
                 b2_ref, g_ref, ys_ref):
    i = pl.program_id(0)

    @pl.when(i < nb_ref[0])
    def _():
        tok = lax.broadcasted_iota(jnp.int32, (BLOCK, S), 1)
        onehot = (tok == srt_ref[...]).astype(jnp.bfloat16)
        xb = jnp.dot(onehot, x2_ref[...],
                     preferred_element_type=jnp.float32).astype(jnp.bfloat16)
        h = jnp.dot(xb, w1_ref[0], preferred_element_type=jnp.float32)
        h = jnp.maximum(h + b1_ref[0], 0.0).astype(jnp.bfloat16)
        o = jnp.dot(h, w2_ref[0], preferred_element_type=jnp.float32)
        ys_ref[...] = (o + b2_ref[0]) * g_ref[...]


def _experts(be, nb, x2b, srt, W1b, b1, W2b, b2, gsrt):
    grid_spec = pltpu.PrefetchScalarGridSpec(
        num_scalar_prefetch=2,
        grid=(NB,),
        in_specs=[
            pl.BlockSpec((S, EMBED), lambda i, be, nb: (0, 0)),
            pl.BlockSpec((BLOCK, 1), lambda i, be, nb: (i, 0)),
            pl.BlockSpec((1, EMBED, HID), lambda i, be, nb: (be[i], 0, 0)),
            pl.BlockSpec((1, 1, HID), lambda i, be, nb: (be[i], 0, 0)),
            pl.BlockSpec((1, HID, EMBED), lambda i, be, nb: (be[i], 0, 0)),
            pl.BlockSpec((1, 1, EMBED), lambda i, be, nb: (be[i], 0, 0)),
            pl.BlockSpec((BLOCK, 1), lambda i, be, nb: (i, 0)),
        ],
        out_specs=pl.BlockSpec((BLOCK, EMBED), lambda i, be, nb: (i, 0)),
    )
    return pl.pallas_call(
        _expert_body,
        grid_spec=grid_spec,
        out_shape=jax.ShapeDtypeStruct((NPAD, EMBED), jnp.float32),
    )(be, nb, x2b, srt.reshape(NPAD, 1), W1b, b1, W2b, b2,
      gsrt.reshape(NPAD, 1))


_TCH = 16


@functools.partial(
    pl.kernel,
    out_type=jax.ShapeDtypeStruct((S, EMBED), jnp.float32),
    mesh=_MESH,
    compiler_params=_SC_PARAMS,
    scratch_types=[
        pltpu.VMEM((2 * _TCH,), jnp.int32),
        pltpu.VMEM((2 * _TCH, EMBED), jnp.float32),
        pltpu.VMEM((_TCH, EMBED), jnp.float32),
        pltpu.SemaphoreType.DMA,
    ],
)
def _combine(ys_hbm, pos_hbm, out_hbm, pidx_v, rows_v, out_v, sem):
    wid = lax.axis_index("s") * 2 + lax.axis_index("c")
    tok_per_w = S // 32

    def step(c, _):
        t0 = wid * tok_per_w + c * _TCH
        pltpu.sync_copy(pos_hbm.at[pl.ds(t0 * 2, 2 * _TCH)], pidx_v)
        pltpu.async_copy(ys_hbm.at[pidx_v], rows_v, sem).wait()
        for j in range(_TCH):
            def vec(v, _, j=j):
                sl = pl.ds(v * 16, 16)
                out_v[j, sl] = rows_v[2 * j, sl] + rows_v[2 * j + 1, sl]
                return 0

            lax.fori_loop(0, EMBED // 16, vec, 0, unroll=8)
        pltpu.sync_copy(out_v, out_hbm.at[pl.ds(t0, _TCH)])
        return 0

    lax.fori_loop(0, tok_per_w // _TCH, step, 0)


def kernel(x, Wr1, br1, Wr2, br2, W1, b1, W2, b2):
    B = x.shape[0]
    x2 = x.reshape(S, EMBED)
    te, tg, cnt = _router(x2, Wr1, br1, Wr2, br2)

    key = te.reshape(P)
    gate = tg.reshape(P)
    srt, gsrt, pos = _compact(key, gate)

    counts = cnt[0]
    nbe = (counts + BLOCK - 1) // BLOCK
    starts = jnp.concatenate(
        [jnp.zeros((1,), jnp.int32), jnp.cumsum(nbe)[:-1].astype(jnp.int32)])
    nblocks = jnp.sum(nbe).astype(jnp.int32)
    ar = jnp.arange(NB, dtype=jnp.int32)
    be = jnp.clip(jnp.sum((starts[None, :] <= ar[:, None]).astype(jnp.int32),
                          axis=1) - 1, 0, NE - 1)
    belast = be[jnp.clip(nblocks - 1, 0, NB - 1)]
    be = jnp.where(ar < nblocks, be, belast).astype(jnp.int32)

    W1b = W1.astype(jnp.bfloat16)
    W2b = W2.astype(jnp.bfloat16)
    ys = _experts(be, nblocks.reshape(1), x2.astype(jnp.bfloat16), srt,
                  W1b, b1.reshape(NE, 1, HID), W2b, b2.reshape(NE, 1, EMBED),
                  gsrt)
    out = _combine(ys, pos)
    return out.reshape(B, S, EMBED)


# speedup vs baseline: 1.0846x; 1.0846x over previous
import functools

import jax
import jax.numpy as jnp
from jax import lax
from jax.experimental import pallas as pl
from jax.experimental.pallas import tpu as pltpu
from jax.experimental.pallas import tpu_sc as plsc

EMBED = 1024
HID = 4096
NE = 8
K = 2
S = 2048
P = S * K
BLOCK = 256
NB = P // BLOCK + NE
NPAD = NB * BLOCK
NEG = -1e30

TB = 256


def _router_body(x_ref, w1_ref, b1_ref, w2_ref, b2_ref,
                 te_ref, tg_ref, cnt_ref):
    i = pl.program_id(0)
    h = jnp.maximum(
        jnp.dot(x_ref[...], w1_ref[...], preferred_element_type=jnp.float32)
        + b1_ref[...], 0.0)
    s = jnp.dot(h, w2_ref[...], preferred_element_type=jnp.float32) + b2_ref[...]
    iota = lax.broadcasted_iota(jnp.int32, (TB, NE), 1)
    m1 = jnp.max(s, axis=1, keepdims=True)
    i1 = jnp.min(jnp.where(s == m1, iota, NE), axis=1, keepdims=True)
    s2 = jnp.where(iota == i1, NEG, s)
    m2 = jnp.max(s2, axis=1, keepdims=True)
    i2 = jnp.min(jnp.where(s2 == m2, iota, NE), axis=1, keepdims=True)
    g1 = 1.0 / (1.0 + jnp.exp(m2 - m1))
    te_ref[...] = jnp.concatenate([i1, i2], axis=1)
    tg_ref[...] = jnp.concatenate([g1, 1.0 - g1], axis=1)
    one = (iota == i1).astype(jnp.int32) + (iota == i2).astype(jnp.int32)
    cadd = jnp.sum(one, axis=0, keepdims=True)

    @pl.when(i == 0)
    def _():
        cnt_ref[...] = cadd

    @pl.when(i > 0)
    def _():
        cnt_ref[...] = cnt_ref[...] + cadd


def _router(x2, Wr1, br1, Wr2, br2):
    return pl.pallas_call(
        _router_body,
        grid=(S // TB,),
        in_specs=[
            pl.BlockSpec((TB, EMBED), lambda i: (i, 0)),
            pl.BlockSpec((EMBED, HID), lambda i: (0, 0)),
            pl.BlockSpec((1, HID), lambda i: (0, 0)),
            pl.BlockSpec((HID, NE), lambda i: (0, 0)),
            pl.BlockSpec((1, NE), lambda i: (0, 0)),
        ],
        out_specs=[
            pl.BlockSpec((TB, K), lambda i: (i, 0)),
            pl.BlockSpec((TB, K), lambda i: (i, 0)),
            pl.BlockSpec((1, NE), lambda i: (0, 0)),
        ],
        out_shape=[
            jax.ShapeDtypeStruct((S, K), jnp.int32),
            jax.ShapeDtypeStruct((S, K), jnp.float32),
            jax.ShapeDtypeStruct((1, NE), jnp.int32),
        ],
    )(x2, Wr1, br1.reshape(1, HID), Wr2, br2.reshape(1, NE))


_MESH = plsc.VectorSubcoreMesh(core_axis_name="c", subcore_axis_name="s")
_SC_PARAMS = pltpu.CompilerParams(needs_layout_passes=False)


@functools.partial(
    pl.kernel,
    out_type=[
        jax.ShapeDtypeStruct((NPAD,), jnp.int32),
        jax.ShapeDtypeStruct((NPAD,), jnp.float32),
    ],
    mesh=_MESH,
    compiler_params=_SC_PARAMS,
    scratch_types=[
        pltpu.VMEM((P,), jnp.int32),
        pltpu.VMEM((P,), jnp.float32),
        pltpu.VMEM((NPAD + 16,), jnp.int32),
        pltpu.VMEM((NPAD + 16,), jnp.float32),
    ],
)
def _compact(key_hbm, g_hbm, srt_hbm, gsrt_hbm,
             key_v, g_v, srt_v, gsrt_v):
    wid = lax.axis_index("s") * 2 + lax.axis_index("c")

    @pl.when(wid == 0)
    def _():
        pltpu.sync_copy(key_hbm, key_v)
        pltpu.sync_copy(g_hbm, g_v)
        zi = jnp.zeros((16,), jnp.int32)
        zf = jnp.zeros((16,), jnp.float32)

        def zero_body(i, _):
            srt_v[pl.ds(i * 16, 16)] = zi
            gsrt_v[pl.ds(i * 16, 16)] = zf
            return 0

        lax.fori_loop(0, NPAD // 16, zero_body, 0)
        lane = lax.iota(jnp.int32, 16)

        ptr = jnp.int32(0)
        for e in range(NE):
            def chunk(c, ptr, e=e):
                pair = c * 16 + lane
                k = key_v[pl.ds(c * 16, 16)]
                g = g_v[pl.ds(c * 16, 16)]
                m = k == e
                cum = jnp.cumsum(m.astype(jnp.int32))
                slots = jnp.where(m, ptr + cum - 1, NPAD + lane)
                plsc.store_scatter(srt_v, [slots], pair >> 1)
                plsc.store_scatter(gsrt_v, [slots], g)
                return ptr + cum[15]

            ptr = lax.fori_loop(0, P // 16, chunk, ptr)
            ptr = ((ptr + BLOCK - 1) // BLOCK) * BLOCK

        pltpu.sync_copy(srt_v.at[pl.ds(0, NPAD)], srt_hbm)
        pltpu.sync_copy(gsrt_v.at[pl.ds(0, NPAD)], gsrt_hbm)


def _expert_body(be_ref, nb_ref, x2_ref, srt_ref, w1_ref, b1_ref, w2_ref,
                 b2_ref, g_ref, out_ref):
    i = pl.program_id(0)

    @pl.when(i == 0)
    def _():
        out_ref[...] = jnp.zeros_like(out_ref)

    @pl.when(i < nb_ref[0])
    def _():
        tok = lax.broadcasted_iota(jnp.int32, (BLOCK, S), 1)
        onehot = (tok == srt_ref[...]).astype(jnp.bfloat16)
        xb = jnp.dot(onehot, x2_ref[...],
                     preferred_element_type=jnp.float32).astype(jnp.bfloat16)
        h = jnp.dot(xb, w1_ref[0], preferred_element_type=jnp.float32)
        h = jnp.maximum(h + b1_ref[0], 0.0).astype(jnp.bfloat16)
        o = jnp.dot(h, w2_ref[0], preferred_element_type=jnp.float32)
        ys = ((o + b2_ref[0]) * g_ref[...]).astype(jnp.bfloat16)
        out_ref[...] += lax.dot_general(
            onehot, ys, (((0,), (0,)), ((), ())),
            preferred_element_type=jnp.float32)


def _experts(be, nb, x2b, srt, W1b, b1, W2b, b2, gsrt):
    grid_spec = pltpu.PrefetchScalarGridSpec(
        num_scalar_prefetch=2,
        grid=(NB,),
        in_specs=[
            pl.BlockSpec((S, EMBED), lambda i, be, nb: (0, 0)),
            pl.BlockSpec((BLOCK, 1), lambda i, be, nb: (i, 0)),
            pl.BlockSpec((1, EMBED, HID), lambda i, be, nb: (be[i], 0, 0)),
            pl.BlockSpec((1, 1, HID), lambda i, be, nb: (be[i], 0, 0)),
            pl.BlockSpec((1, HID, EMBED), lambda i, be, nb: (be[i], 0, 0)),
            pl.BlockSpec((1, 1, EMBED), lambda i, be, nb: (be[i], 0, 0)),
            pl.BlockSpec((BLOCK, 1), lambda i, be, nb: (i, 0)),
        ],
        out_specs=pl.BlockSpec((S, EMBED), lambda i, be, nb: (0, 0)),
    )
    return pl.pallas_call(
        _expert_body,
        grid_spec=grid_spec,
        out_shape=jax.ShapeDtypeStruct((S, EMBED), jnp.float32),
        compiler_params=pltpu.CompilerParams(
            vmem_limit_bytes=110 * 1024 * 1024),
    )(be, nb, x2b, srt.reshape(NPAD, 1), W1b, b1, W2b, b2,
      gsrt.reshape(NPAD, 1))


def kernel(x, Wr1, br1, Wr2, br2, W1, b1, W2, b2):
    B = x.shape[0]
    x2 = x.reshape(S, EMBED)
    te, tg, cnt = _router(x2, Wr1, br1, Wr2, br2)

    key = te.reshape(P)
    gate = tg.reshape(P)
    srt, gsrt = _compact(key, gate)

    counts = cnt[0]
    nbe = (counts + BLOCK - 1) // BLOCK
    starts = jnp.concatenate(
        [jnp.zeros((1,), jnp.int32), jnp.cumsum(nbe)[:-1].astype(jnp.int32)])
    nblocks = jnp.sum(nbe).astype(jnp.int32)
    ar = jnp.arange(NB, dtype=jnp.int32)
    be = jnp.clip(jnp.sum((starts[None, :] <= ar[:, None]).astype(jnp.int32),
                          axis=1) - 1, 0, NE - 1)
    belast = be[jnp.clip(nblocks - 1, 0, NB - 1)]
    be = jnp.where(ar < nblocks, be, belast).astype(jnp.int32)

    W1b = W1.astype(jnp.bfloat16)
    W2b = W2.astype(jnp.bfloat16)
    out = _experts(be, nblocks.reshape(1), x2.astype(jnp.bfloat16), srt,
                   W1b, b1.reshape(NE, 1, HID), W2b, b2.reshape(NE, 1, EMBED),
                   gsrt)
    return out.reshape(B, S, EMBED)
